# baseline (speedup 1.0000x reference)
import functools

import jax
import jax.numpy as jnp
from jax import lax
from jax.experimental import pallas as pl
from jax.experimental.pallas import tpu as pltpu
from jax.experimental.pallas import tpu_sc as plsc

VOCAB = 1000000
EMB = 128
OUT = 64
BATCH = 16384
HIST = 50
B_TOTAL = BATCH * HIST

NC, NS = 2, 16
NW = NC * NS
CH = 128
NBUF = 5

JBLK = NW * CH
PROJ_BB = 16384
TCJ = PROJ_BB // JBLK
NJ = B_TOTAL // JBLK
NB = BATCH // PROJ_BB

SLICES = [20, 40, 40, 40, 40, 20]
assert sum(SLICES) == NJ and all(n % NBUF == 0 for n in SLICES)


@functools.cache
def _make_gather(nchunk):
    ngrp = nchunk // NBUF
    b_per_w = nchunk * CH
    mesh = plsc.VectorSubcoreMesh(core_axis_name="c", subcore_axis_name="s")

    @functools.partial(
        pl.kernel,
        mesh=mesh,
        out_type=jax.ShapeDtypeStruct((NW * b_per_w, EMB), jnp.float32),
        scratch_types=[
            pltpu.VMEM((nchunk, CH), jnp.int32),
            pltpu.VMEM((NBUF, CH, EMB), jnp.float32),
        ] + [pltpu.SemaphoreType.DMA] * (2 * NBUF),
    )
    def gather_k(tab_hbm, idx_hbm, out_hbm, idx_v, rows_v, *sems):
        gsems, osems = sems[:NBUF], sems[NBUF:]
        wid = lax.axis_index("s") * NC + lax.axis_index("c")
        base = wid * b_per_w
        pltpu.sync_copy(idx_hbm.at[wid], idx_v)

        def gather_copy(c, b):
            return pltpu.make_async_copy(
                tab_hbm.at[idx_v.at[c]], rows_v.at[b], gsems[b])

        def out_copy(c, b):
            return pltpu.make_async_copy(
                rows_v.at[b], out_hbm.at[pl.ds(base + c * CH, CH)], osems[b])

        for b in range(NBUF):
            gather_copy(b, b).start()

        def group(g, carry):
            for b in range(NBUF):
                c = g * NBUF + b
                gather_copy(c, b).wait()
                out_copy(c, b).start()
            for b in range(NBUF):
                c = g * NBUF + b
                out_copy(c, b).wait()

                @pl.when(g < ngrp - 1)
                def _():
                    gather_copy(c + NBUF, b).start()

            return carry

        lax.fori_loop(0, ngrp, group, 0)

    return gather_k


def _project_slice(gathered, W, j0, nj, prev=None):
    def body(w_ref, g_ref, *refs):
        r = lax.dot_general(
            w_ref[...], g_ref[...],
            dimension_numbers=(((1,), (1,)), ((), ())),
            preferred_element_type=jnp.float32,
        )
        refs[-1][...] = r.reshape(1, OUT, PROJ_BB)

    in_specs = [
        pl.BlockSpec((OUT, EMB), lambda j: (0, 0)),
        pl.BlockSpec((PROJ_BB, EMB), lambda j: (j, 0)),
    ]
    args = [W, gathered]
    kwargs = {}
    if prev is not None:
        in_specs.append(pl.BlockSpec(memory_space=pl.ANY))
        args.append(prev)
        kwargs["input_output_aliases"] = {2: 0}

    jt0 = j0 // TCJ
    return pl.pallas_call(
        body,
        grid=(nj // TCJ,),
        in_specs=in_specs,
        out_specs=pl.BlockSpec(
            (1, OUT, PROJ_BB),
            lambda j: ((j + jt0) // NB, 0, (j + jt0) % NB)),
        out_shape=jax.ShapeDtypeStruct((HIST, OUT, BATCH), jnp.float32),
        **kwargs,
    )(*args)


def kernel(seqs, emb_table, W):
    idx = seqs.astype(jnp.int32).T.reshape(-1)
    out = None
    j0 = 0
    for nj in SLICES:
        idx_s = lax.slice(idx, (j0 * JBLK,), ((j0 + nj) * JBLK,))
        gathered = _make_gather(nj)(
            emb_table, idx_s.reshape(NW, nj, CH))
        out = _project_slice(gathered, W, j0, nj, out)
        j0 += nj
    return out.transpose(2, 0, 1)


# speedup vs baseline: 1.0256x; 1.0256x over previous
import functools

import jax
import jax.numpy as jnp
from jax import lax
from jax.experimental import pallas as pl
from jax.experimental.pallas import tpu as pltpu
from jax.experimental.pallas import tpu_sc as plsc

VOCAB = 1000000
EMB = 128
OUT = 64
BATCH = 16384
HIST = 50
B_TOTAL = BATCH * HIST

NC, NS = 2, 16
NW = NC * NS
CH = 64
NBUF = 10

JBLK = 4096
PROJ_BB = 16384
TCJ = PROJ_BB // JBLK
NJ = B_TOTAL // JBLK
NB = BATCH // PROJ_BB

SLICES = [40, 40, 40, 40, 40]
assert sum(SLICES) == NJ and all(
    (n * JBLK // (NW * CH)) % NBUF == 0 and n % TCJ == 0 for n in SLICES)


@functools.cache
def _make_gather(nchunk):
    ngrp = nchunk // NBUF
    b_per_w = nchunk * CH
    mesh = plsc.VectorSubcoreMesh(core_axis_name="c", subcore_axis_name="s")

    @functools.partial(
        pl.kernel,
        mesh=mesh,
        out_type=jax.ShapeDtypeStruct((NW * b_per_w, EMB), jnp.float32),
        scratch_types=[
            pltpu.VMEM((nchunk, CH), jnp.int32),
            pltpu.VMEM((NBUF, CH, EMB), jnp.float32),
        ] + [pltpu.SemaphoreType.DMA] * (2 * NBUF),
    )
    def gather_k(tab_hbm, idx_hbm, out_hbm, idx_v, rows_v, *sems):
        gsems, osems = sems[:NBUF], sems[NBUF:]
        wid = lax.axis_index("s") * NC + lax.axis_index("c")
        base = wid * b_per_w
        pltpu.sync_copy(idx_hbm.at[wid], idx_v)

        def gather_copy(c, b):
            return pltpu.make_async_copy(
                tab_hbm.at[idx_v.at[c]], rows_v.at[b], gsems[b])

        def out_copy(c, b):
            return pltpu.make_async_copy(
                rows_v.at[b], out_hbm.at[pl.ds(base + c * CH, CH)], osems[b])

        for b in range(NBUF):
            gather_copy(b, b).start()

        def group(g, carry):
            for b in range(NBUF):
                c = g * NBUF + b
                gather_copy(c, b).wait()
                out_copy(c, b).start()
            for b in range(NBUF):
                c = g * NBUF + b
                out_copy(c, b).wait()

                @pl.when(g < ngrp - 1)
                def _():
                    gather_copy(c + NBUF, b).start()

            return carry

        lax.fori_loop(0, ngrp, group, 0)

    return gather_k


def _project_slice(gathered, W, j0, nj, prev=None):
    def body(w_ref, g_ref, *refs):
        r = lax.dot_general(
            w_ref[...], g_ref[...],
            dimension_numbers=(((1,), (1,)), ((), ())),
            preferred_element_type=jnp.float32,
        )
        refs[-1][...] = r.reshape(1, OUT, PROJ_BB)

    in_specs = [
        pl.BlockSpec((OUT, EMB), lambda j: (0, 0)),
        pl.BlockSpec((PROJ_BB, EMB), lambda j: (j, 0)),
    ]
    args = [W, gathered]
    kwargs = {}
    if prev is not None:
        in_specs.append(pl.BlockSpec(memory_space=pl.ANY))
        args.append(prev)
        kwargs["input_output_aliases"] = {2: 0}

    jt0 = j0 // TCJ
    return pl.pallas_call(
        body,
        grid=(nj // TCJ,),
        in_specs=in_specs,
        out_specs=pl.BlockSpec(
            (1, OUT, PROJ_BB),
            lambda j: ((j + jt0) // NB, 0, (j + jt0) % NB)),
        out_shape=jax.ShapeDtypeStruct((HIST, OUT, BATCH), jnp.float32),
        **kwargs,
    )(*args)


def kernel(seqs, emb_table, W):
    idx = seqs.astype(jnp.int32).T.reshape(-1)
    out = None
    j0 = 0
    for nj in SLICES:
        idx_s = lax.slice(idx, (j0 * JBLK,), ((j0 + nj) * JBLK,))
        nchunk = nj * JBLK // (NW * CH)
        gathered = _make_gather(nchunk)(
            emb_table, idx_s.reshape(NW, nchunk, CH))
        out = _project_slice(gathered, W, j0, nj, out)
        j0 += nj
    return out.transpose(2, 0, 1)
